# baseline (speedup 1.0000x reference)
import jax
import jax.numpy as jnp
from jax import lax
from jax.experimental import pallas as pl
from jax.experimental.pallas import tpu as pltpu

_H = 24
_W = 24


def _pooled_rows(x):
    c, hw = x.shape
    hm = jnp.mean(x, axis=0, keepdims=True)
    idx = lax.broadcasted_iota(jnp.int32, (1, hw), 1)
    h = idx // _W
    w = idx - h * _W
    mx = hm
    mn = hm
    for dh in (-1, 0, 1):
        for dw in (-1, 0, 1):
            if dh == 0 and dw == 0:
                continue
            s = dh * _W + dw
            r = jnp.concatenate([hm[:, s:], hm[:, :s]], axis=1)
            hn = h + dh
            wn = w + dw
            valid = (hn >= 0) & (hn < _H) & (wn >= 0) & (wn < _W)
            mx = jnp.maximum(mx, jnp.where(valid, r, -jnp.inf))
            mn = jnp.minimum(mn, jnp.where(valid, r, jnp.inf))
    corner = ((h == 0) | (h == _H - 1)) & ((w == 0) | (w == _W - 1))
    not_corner = jnp.logical_not(corner)
    mask_max = ((hm >= mx) & not_corner).astype(jnp.float32)
    mask_min = ((hm <= mn) & not_corner).astype(jnp.float32)
    cmax = jnp.sum(mask_max, axis=1, keepdims=True)
    cmin = jnp.sum(mask_min, axis=1, keepdims=True)
    rows = jnp.concatenate(
        [
            mask_max / cmax,
            mask_min / cmin,
            jnp.full((1, hw), 1.0 / hw, dtype=jnp.float32),
            jnp.zeros((5, hw), dtype=jnp.float32),
        ],
        axis=0,
    )
    wt = jnp.transpose(rows)
    f = lax.dot_general(
        x, wt, (((1,), (0,)), ((), ())), preferred_element_type=jnp.float32
    )
    return jnp.transpose(f)


def _pool_kernel(x1_ref, x2_ref, e1_ref, e2_ref):
    e1_ref[0] = _pooled_rows(x1_ref[0])
    e2_ref[0] = _pooled_rows(x2_ref[0])


def _loss_kernel(f1_ref, f2_ref, s_ref, out_ref):
    f1 = f1_ref[...]
    f2 = f2_ref[...]
    b = f1.shape[0]
    scale = s_ref[0]

    nsq1 = jnp.zeros((b, 1), dtype=jnp.float32)
    nsq2 = jnp.zeros((b, 1), dtype=jnp.float32)
    lraw = jnp.zeros((b, b), dtype=jnp.float32)
    for k in range(3):
        a = f1[:, k, :]
        bb = f2[:, k, :]
        nsq1 = nsq1 + jnp.sum(a * a, axis=1, keepdims=True)
        nsq2 = nsq2 + jnp.sum(bb * bb, axis=1, keepdims=True)
        lraw = lraw + lax.dot_general(
            a, bb, (((1,), (1,)), ((), ())), preferred_element_type=jnp.float32
        )
    rn1 = lax.rsqrt(nsq1)
    rn2_row = jnp.transpose(lax.rsqrt(nsq2))
    logits = (scale * lraw) * rn1 * rn2_row

    m_r = jnp.max(logits, axis=1, keepdims=True)
    lse_r = jnp.log(jnp.sum(jnp.exp(logits - m_r), axis=1, keepdims=True)) + m_r
    m_c = jnp.max(logits, axis=0, keepdims=True)
    lse_c = jnp.log(jnp.sum(jnp.exp(logits - m_c), axis=0, keepdims=True)) + m_c

    ii = lax.broadcasted_iota(jnp.int32, (b, b), 0)
    jj = lax.broadcasted_iota(jnp.int32, (b, b), 1)
    diag_sum = jnp.sum(
        jnp.where(ii == jj, logits, 0.0), axis=0, keepdims=True
    )
    diag_sum = jnp.sum(diag_sum, axis=1, keepdims=True)

    s_r = jnp.sum(lse_r, axis=0, keepdims=True)
    s_c = jnp.sum(lse_c, axis=1, keepdims=True)
    out_ref[...] = (s_r + s_c) / (2.0 * b) - diag_sum / b


def kernel(image_features1, image_features2, logit_scale):
    b, c, h, w = image_features1.shape
    hw = h * w
    x1 = image_features1.reshape(b, c, hw)
    x2 = image_features2.reshape(b, c, hw)

    f1, f2 = pl.pallas_call(
        _pool_kernel,
        grid=(b,),
        in_specs=[
            pl.BlockSpec((1, c, hw), lambda i: (i, 0, 0)),
            pl.BlockSpec((1, c, hw), lambda i: (i, 0, 0)),
        ],
        out_specs=[
            pl.BlockSpec((1, 8, c), lambda i: (i, 0, 0)),
            pl.BlockSpec((1, 8, c), lambda i: (i, 0, 0)),
        ],
        out_shape=[
            jax.ShapeDtypeStruct((b, 8, c), jnp.float32),
            jax.ShapeDtypeStruct((b, 8, c), jnp.float32),
        ],
        compiler_params=pltpu.CompilerParams(
            dimension_semantics=("parallel",),
        ),
    )(x1, x2)

    loss = pl.pallas_call(
        _loss_kernel,
        in_specs=[
            pl.BlockSpec((b, 8, c), lambda: (0, 0, 0)),
            pl.BlockSpec((b, 8, c), lambda: (0, 0, 0)),
            pl.BlockSpec(memory_space=pltpu.SMEM),
        ],
        out_specs=pl.BlockSpec((1, 1), lambda: (0, 0)),
        out_shape=jax.ShapeDtypeStruct((1, 1), jnp.float32),
    )(f1, f2, logit_scale)

    return loss.reshape(())
